# baseline (speedup 1.0000x reference)
import jax
import jax.numpy as jnp
from jax import lax
from jax.experimental import pallas as pl
from jax.experimental.pallas import tpu as pltpu
from jax.experimental.pallas import tpu_sc as plsc

_THRES = 0.5
_L = 4096
_A = 2048
_BR = 512
_BW = 1024
_NEG = float("-inf")

_NW = 32
_SCROWS = _L - _A
_RPW = _SCROWS // _NW
_BLK = 8
_NBLK = _RPW // _BLK


def _sc_reduce(x_hbm, pmax_hbm, pidx_hbm, bufa, bufb, pma, pmb, pia, pib,
               sina, sinb, souta, soutb):
    c = lax.axis_index("c")
    s = lax.axis_index("s")
    wid = s * 2 + c
    r0 = _A + wid * _RPW
    iota16 = lax.iota(jnp.int32, 16)
    bufs, pms, pis = (bufa, bufb), (pma, pmb), (pia, pib)
    sins, souts = (sina, sinb), (souta, soutb)

    cin = [None, None]
    cout = [None, None]
    cin[0] = pltpu.async_copy(x_hbm.at[pl.ds(r0, _BLK)], bufs[0], sins[0])
    for k in range(_NBLK):
        b = k % 2
        if k + 1 < _NBLK:
            nb = (k + 1) % 2
            cin[nb] = pltpu.async_copy(
                x_hbm.at[pl.ds(r0 + (k + 1) * _BLK, _BLK)], bufs[nb], sins[nb])
        cin[b].wait()
        if cout[b] is not None:
            cout[b][0].wait()
            cout[b][1].wait()
        buf, pm, pi = bufs[b], pms[b], pis[b]
        for rr in range(_BLK):
            grow = r0 + k * _BLK + rr
            qb = pl.multiple_of((grow // 16) * 16, 16)
            v = buf[rr, pl.ds(qb, 16)]
            v = jnp.where(iota16 == (grow % 16), jnp.float32(_NEG), v)
            buf[rr, pl.ds(qb, 16)] = v
        for rr in range(0, _BLK, 4):
            def _qbody(q, st):
                m0, i0, m1, i1, m2, i2, m3, i3, jv = st
                v0 = buf[rr, pl.ds(q * 16, 16)]
                v1 = buf[rr + 1, pl.ds(q * 16, 16)]
                v2 = buf[rr + 2, pl.ds(q * 16, 16)]
                v3 = buf[rr + 3, pl.ds(q * 16, 16)]
                i0 = jnp.where(v0 > m0, jv, i0)
                m0 = jnp.maximum(m0, v0)
                i1 = jnp.where(v1 > m1, jv, i1)
                m1 = jnp.maximum(m1, v1)
                i2 = jnp.where(v2 > m2, jv, i2)
                m2 = jnp.maximum(m2, v2)
                i3 = jnp.where(v3 > m3, jv, i3)
                m3 = jnp.maximum(m3, v3)
                return (m0, i0, m1, i1, m2, i2, m3, i3, jv + 16)
            mneg = jnp.full((16,), _NEG, jnp.float32)
            izero = jnp.full((16,), 0, jnp.int32)
            st = lax.fori_loop(
                0, _L // 16, _qbody,
                (mneg, izero, mneg, izero, mneg, izero, mneg, izero, iota16),
                unroll=4)
            for t in range(4):
                pm[rr + t, pl.ds(0, 16)] = st[2 * t]
                pi[rr + t, pl.ds(0, 16)] = st[2 * t + 1]
        row0 = k * _BLK + wid * _RPW
        cout[b] = (
            pltpu.async_copy(pm, pmax_hbm.at[pl.ds(row0, _BLK)], souts[b]),
            pltpu.async_copy(pi, pidx_hbm.at[pl.ds(row0, _BLK)], souts[b]),
        )
    for b in range(2):
        if cout[b] is not None:
            cout[b][0].wait()
            cout[b][1].wait()


def _reduce_body(x_ref, c2_ref, d2_ref):
    i = pl.program_id(0)
    r0 = i * _BR
    x = x_ref[...]
    rows = jax.lax.broadcasted_iota(jnp.int32, (_BR, _L), 0) + r0
    cols = jax.lax.broadcasted_iota(jnp.int32, (_BR, _L), 1)
    xm = jnp.where(rows == cols, _NEG, x)
    vals = jnp.max(xm, axis=1)
    ismax = xm == vals[:, None]
    inds = jnp.min(jnp.where(ismax, cols, _L), axis=1)
    msk = vals > _THRES
    rowid = jax.lax.iota(jnp.int32, _BR) + r0
    c2_ref[...] = jnp.where(msk, inds, rowid)[:, None]
    d2_ref[...] = jnp.where(msk, inds, -1)[None, :]


def _fin_asm_body(tc_c2_ref, tc_d2_ref, pmax_ref, pidx_ref, out_ref,
                  c2s_ref, d2s_ref):
    i = pl.program_id(0)

    @pl.when(i == 0)
    def _finalize():
        pm = pmax_ref[...]
        pi = pidx_ref[...]
        vals = jnp.max(pm, axis=1)
        ismax = pm == vals[:, None]
        inds = jnp.min(jnp.where(ismax, pi, _L), axis=1)
        msk = vals > _THRES
        rowid = jax.lax.iota(jnp.int32, _SCROWS) + _A
        c2s_ref[pl.ds(0, _A), :] = tc_c2_ref[...]
        d2s_ref[:, pl.ds(0, _A)] = tc_d2_ref[...]
        c2s_ref[pl.ds(_A, _SCROWS), :] = jnp.where(msk, inds, rowid)[:, None]
        d2s_ref[:, pl.ds(_A, _SCROWS)] = jnp.where(msk, inds, -1)[None, :]

    r0 = i * _BW
    d2_row = d2s_ref[...]
    c2_col = c2s_ref[pl.ds(r0, _BW), :]
    rows = jax.lax.broadcasted_iota(jnp.int32, (_BW, _L), 0) + r0
    cols = jax.lax.broadcasted_iota(jnp.int32, (_BW, _L), 1)
    hit = (rows == cols) | (c2_col == cols) | (d2_row == rows)
    out_ref[...] = jnp.where(hit, jnp.float32(1.0), jnp.float32(0.0))


def kernel(input):
    x2d = input.reshape(_L, _L)

    mesh = plsc.VectorSubcoreMesh(core_axis_name="c", subcore_axis_name="s")
    pmax, pidx = pl.kernel(
        _sc_reduce,
        cost_estimate=pl.CostEstimate(
            flops=3 * _SCROWS * _L, bytes_accessed=4 * _SCROWS * _L,
            transcendentals=0),
        out_type=[
            jax.ShapeDtypeStruct((_SCROWS, 16), jnp.float32),
            jax.ShapeDtypeStruct((_SCROWS, 16), jnp.int32),
        ],
        mesh=mesh,
        scratch_types=[
            pltpu.VMEM((_BLK, _L), jnp.float32),
            pltpu.VMEM((_BLK, _L), jnp.float32),
            pltpu.VMEM((_BLK, 16), jnp.float32),
            pltpu.VMEM((_BLK, 16), jnp.float32),
            pltpu.VMEM((_BLK, 16), jnp.int32),
            pltpu.VMEM((_BLK, 16), jnp.int32),
            pltpu.SemaphoreType.DMA,
            pltpu.SemaphoreType.DMA,
            pltpu.SemaphoreType.DMA,
            pltpu.SemaphoreType.DMA,
        ],
    )(x2d)

    g1 = _A // _BR
    tc_c2, tc_d2 = pl.pallas_call(
        _reduce_body,
        grid=(g1,),
        cost_estimate=pl.CostEstimate(
            flops=5 * _A * _L, bytes_accessed=4 * _A * _L, transcendentals=0),
        in_specs=[pl.BlockSpec((_BR, _L), lambda i: (i, 0))],
        out_specs=[
            pl.BlockSpec((_BR, 1), lambda i: (i, 0)),
            pl.BlockSpec((1, _BR), lambda i: (0, i)),
        ],
        out_shape=[
            jax.ShapeDtypeStruct((_A, 1), jnp.int32),
            jax.ShapeDtypeStruct((1, _A), jnp.int32),
        ],
    )(x2d)

    g2 = _L // _BW
    out2d = pl.pallas_call(
        _fin_asm_body,
        grid=(g2,),
        in_specs=[
            pl.BlockSpec((_A, 1), lambda i: (0, 0)),
            pl.BlockSpec((1, _A), lambda i: (0, 0)),
            pl.BlockSpec((_SCROWS, 16), lambda i: (0, 0)),
            pl.BlockSpec((_SCROWS, 16), lambda i: (0, 0)),
        ],
        out_specs=pl.BlockSpec((_BW, _L), lambda i: (i, 0)),
        out_shape=jax.ShapeDtypeStruct((_L, _L), jnp.float32),
        scratch_shapes=[
            pltpu.VMEM((_L, 1), jnp.int32),
            pltpu.VMEM((1, _L), jnp.int32),
        ],
    )(tc_c2, tc_d2, pmax, pidx)
    return out2d.reshape(input.shape)


# speedup vs baseline: 1.0476x; 1.0476x over previous
import jax
import jax.numpy as jnp
from jax import lax
from jax.experimental import pallas as pl
from jax.experimental.pallas import tpu as pltpu
from jax.experimental.pallas import tpu_sc as plsc

_THRES = 0.5
_L = 4096
_A = 3072
_BR = 1024
_BW = 512
_NEG = float("-inf")

_NW = 32
_SCROWS = _L - _A
_RPW = _SCROWS // _NW
_BLK = 8
_NBLK = _RPW // _BLK


def _sc_reduce(x_hbm, pmax_hbm, pidx_hbm, bufa, bufb, pma, pmb, pia, pib,
               sina, sinb, souta, soutb):
    c = lax.axis_index("c")
    s = lax.axis_index("s")
    wid = s * 2 + c
    r0 = _A + wid * _RPW
    iota16 = lax.iota(jnp.int32, 16)
    bufs, pms, pis = (bufa, bufb), (pma, pmb), (pia, pib)
    sins, souts = (sina, sinb), (souta, soutb)

    cin = [None, None]
    cout = [None, None]
    cin[0] = pltpu.async_copy(x_hbm.at[pl.ds(r0, _BLK)], bufs[0], sins[0])
    for k in range(_NBLK):
        b = k % 2
        if k + 1 < _NBLK:
            nb = (k + 1) % 2
            cin[nb] = pltpu.async_copy(
                x_hbm.at[pl.ds(r0 + (k + 1) * _BLK, _BLK)], bufs[nb], sins[nb])
        cin[b].wait()
        if cout[b] is not None:
            cout[b][0].wait()
            cout[b][1].wait()
        buf, pm, pi = bufs[b], pms[b], pis[b]
        for rr in range(_BLK):
            grow = r0 + k * _BLK + rr
            qb = pl.multiple_of((grow // 16) * 16, 16)
            v = buf[rr, pl.ds(qb, 16)]
            v = jnp.where(iota16 == (grow % 16), jnp.float32(_NEG), v)
            buf[rr, pl.ds(qb, 16)] = v
        for rr in range(0, _BLK, 4):
            def _qbody(q, st):
                m0, i0, m1, i1, m2, i2, m3, i3, jv = st
                v0 = buf[rr, pl.ds(q * 16, 16)]
                v1 = buf[rr + 1, pl.ds(q * 16, 16)]
                v2 = buf[rr + 2, pl.ds(q * 16, 16)]
                v3 = buf[rr + 3, pl.ds(q * 16, 16)]
                i0 = jnp.where(v0 > m0, jv, i0)
                m0 = jnp.maximum(m0, v0)
                i1 = jnp.where(v1 > m1, jv, i1)
                m1 = jnp.maximum(m1, v1)
                i2 = jnp.where(v2 > m2, jv, i2)
                m2 = jnp.maximum(m2, v2)
                i3 = jnp.where(v3 > m3, jv, i3)
                m3 = jnp.maximum(m3, v3)
                return (m0, i0, m1, i1, m2, i2, m3, i3, jv + 16)
            mneg = jnp.full((16,), _NEG, jnp.float32)
            izero = jnp.full((16,), 0, jnp.int32)
            st = lax.fori_loop(
                0, _L // 16, _qbody,
                (mneg, izero, mneg, izero, mneg, izero, mneg, izero, iota16),
                unroll=4)
            for t in range(4):
                pm[rr + t, pl.ds(0, 16)] = st[2 * t]
                pi[rr + t, pl.ds(0, 16)] = st[2 * t + 1]
        row0 = k * _BLK + wid * _RPW
        cout[b] = (
            pltpu.async_copy(pm, pmax_hbm.at[pl.ds(row0, _BLK)], souts[b]),
            pltpu.async_copy(pi, pidx_hbm.at[pl.ds(row0, _BLK)], souts[b]),
        )
    for b in range(2):
        if cout[b] is not None:
            cout[b][0].wait()
            cout[b][1].wait()


def _reduce_body(x_ref, c2_ref, d2_ref):
    i = pl.program_id(0)
    r0 = i * _BR
    x = x_ref[...]
    rows = jax.lax.broadcasted_iota(jnp.int32, (_BR, _L), 0) + r0
    cols = jax.lax.broadcasted_iota(jnp.int32, (_BR, _L), 1)
    xm = jnp.where(rows == cols, _NEG, x)
    vals = jnp.max(xm, axis=1)
    ismax = xm == vals[:, None]
    inds = jnp.min(jnp.where(ismax, cols, _L), axis=1)
    msk = vals > _THRES
    rowid = jax.lax.iota(jnp.int32, _BR) + r0
    c2_ref[...] = jnp.where(msk, inds, rowid)[:, None]
    d2_ref[...] = jnp.where(msk, inds, -1)[None, :]


def _fin_asm_body(tc_c2_ref, tc_d2_ref, pmax_ref, pidx_ref, out_ref,
                  c2s_ref, d2s_ref):
    i = pl.program_id(0)

    @pl.when(i == 0)
    def _finalize():
        pm = pmax_ref[...]
        pi = pidx_ref[...]
        vals = jnp.max(pm, axis=1)
        ismax = pm == vals[:, None]
        inds = jnp.min(jnp.where(ismax, pi, _L), axis=1)
        msk = vals > _THRES
        rowid = jax.lax.iota(jnp.int32, _SCROWS) + _A
        c2s_ref[pl.ds(0, _A), :] = tc_c2_ref[...]
        d2s_ref[:, pl.ds(0, _A)] = tc_d2_ref[...]
        c2s_ref[pl.ds(_A, _SCROWS), :] = jnp.where(msk, inds, rowid)[:, None]
        d2s_ref[:, pl.ds(_A, _SCROWS)] = jnp.where(msk, inds, -1)[None, :]

    r0 = i * _BW
    d2_row = d2s_ref[...]
    c2_col = c2s_ref[pl.ds(r0, _BW), :]
    rows = jax.lax.broadcasted_iota(jnp.int32, (_BW, _L), 0) + r0
    cols = jax.lax.broadcasted_iota(jnp.int32, (_BW, _L), 1)
    hit = (rows == cols) | (c2_col == cols) | (d2_row == rows)
    out_ref[...] = jnp.where(hit, jnp.float32(1.0), jnp.float32(0.0))


def kernel(input):
    x2d = input.reshape(_L, _L)

    mesh = plsc.VectorSubcoreMesh(core_axis_name="c", subcore_axis_name="s")
    pmax, pidx = pl.kernel(
        _sc_reduce,
        cost_estimate=pl.CostEstimate(
            flops=3 * _SCROWS * _L, bytes_accessed=4 * _SCROWS * _L,
            transcendentals=0),
        out_type=[
            jax.ShapeDtypeStruct((_SCROWS, 16), jnp.float32),
            jax.ShapeDtypeStruct((_SCROWS, 16), jnp.int32),
        ],
        mesh=mesh,
        scratch_types=[
            pltpu.VMEM((_BLK, _L), jnp.float32),
            pltpu.VMEM((_BLK, _L), jnp.float32),
            pltpu.VMEM((_BLK, 16), jnp.float32),
            pltpu.VMEM((_BLK, 16), jnp.float32),
            pltpu.VMEM((_BLK, 16), jnp.int32),
            pltpu.VMEM((_BLK, 16), jnp.int32),
            pltpu.SemaphoreType.DMA,
            pltpu.SemaphoreType.DMA,
            pltpu.SemaphoreType.DMA,
            pltpu.SemaphoreType.DMA,
        ],
    )(x2d)

    g1 = _A // _BR
    tc_c2, tc_d2 = pl.pallas_call(
        _reduce_body,
        grid=(g1,),
        cost_estimate=pl.CostEstimate(
            flops=5 * _A * _L, bytes_accessed=4 * _A * _L, transcendentals=0),
        in_specs=[pl.BlockSpec((_BR, _L), lambda i: (i, 0))],
        out_specs=[
            pl.BlockSpec((_BR, 1), lambda i: (i, 0)),
            pl.BlockSpec((1, _BR), lambda i: (0, i)),
        ],
        out_shape=[
            jax.ShapeDtypeStruct((_A, 1), jnp.int32),
            jax.ShapeDtypeStruct((1, _A), jnp.int32),
        ],
    )(x2d)

    g2 = _L // _BW
    out2d = pl.pallas_call(
        _fin_asm_body,
        grid=(g2,),
        in_specs=[
            pl.BlockSpec((_A, 1), lambda i: (0, 0)),
            pl.BlockSpec((1, _A), lambda i: (0, 0)),
            pl.BlockSpec((_SCROWS, 16), lambda i: (0, 0)),
            pl.BlockSpec((_SCROWS, 16), lambda i: (0, 0)),
        ],
        out_specs=pl.BlockSpec((_BW, _L), lambda i: (i, 0)),
        out_shape=jax.ShapeDtypeStruct((_L, _L), jnp.float32),
        scratch_shapes=[
            pltpu.VMEM((_L, 1), jnp.int32),
            pltpu.VMEM((1, _L), jnp.int32),
        ],
    )(tc_c2, tc_d2, pmax, pidx)
    return out2d.reshape(input.shape)
